# baseline (speedup 1.0000x reference)
import functools

import jax
import jax.numpy as jnp
from jax import lax
from jax.experimental import pallas as pl
from jax.experimental.pallas import tpu as pltpu
from jax.experimental.pallas import tpu_sc as plsc

_SMOOTH_W = 0.0008
_SPARSE_W = 0.0008

_L = 16
_NC = 2
_NS = 16
_NW = _NC * _NS
_B = 128
_N = 32768
_RPW = _B // _NW
_S = _N // _L
_SEG = _S + 1
_MAIN = 2032
_WIN = _SEG * (_L - 1) + 9
_UNROLL = 2

_BAG_SCALE = -1.0 / 3.0


def _sigmoid_from_z(z):
    return 1.0 / (1.0 + jnp.exp(z))


def _log1p_newton(z):
    w = 1.0 + z
    t = z * (1.0 - z * (0.5 - z * (1.0 / 3.0)))
    for _ in range(3):
        t = t - 1.0 + w * jnp.exp(-t)
    return t


def _body(logits_hbm, label_hbm, out_hbm, buf0, buf1, label_v, out_v,
          sem0, sem1, lsem):
    cid = lax.axis_index("c")
    sid = lax.axis_index("s")
    wid = sid * _NC + cid
    lanes = lax.iota(jnp.int32, _L)
    b_vec = lanes * _SEG

    bufs = [buf0, buf1]
    sems = [sem0, sem1]
    row0 = wid * _RPW
    copies = [pltpu.async_copy(logits_hbm.at[row0], bufs[0], sems[0])]
    label_copy = pltpu.async_copy(label_hbm, label_v, lsem)

    pos_inf = jnp.full((_L,), jnp.inf, dtype=jnp.float32)
    zeros = jnp.zeros((_L,), jnp.float32)

    bag_acc = zeros
    sparse_acc = zeros
    smooth_acc = zeros

    for j in range(_RPW):
        if j + 1 < _RPW:
            copies.append(pltpu.async_copy(
                logits_hbm.at[row0 + j + 1], bufs[(j + 1) % 2],
                sems[(j + 1) % 2]))
        copies[j].wait()
        rbuf = bufs[j % 2]

        z_prev = -plsc.load_gather(rbuf, [jnp.maximum(b_vec - 1, 0)])
        s_prev = _sigmoid_from_z(z_prev)

        @plsc.parallel_loop(
            0, _MAIN, step=8, unroll=_UNROLL,
            carry=(pos_inf, pos_inf, pos_inf, sparse_acc, smooth_acc,
                   s_prev))
        def _row_loop(i, carry, rbuf=rbuf):
            t1, t2, t3, sp, sm, spv = carry
            win = rbuf.at[pl.ds(i, _WIN)]
            for u in range(8):
                x = plsc.load_gather(win, [b_vec + u])
                z = -x
                m1 = jnp.maximum(t1, z)
                t1 = jnp.minimum(t1, z)
                m2 = jnp.maximum(t2, m1)
                t2 = jnp.minimum(t2, m1)
                t3 = jnp.minimum(t3, m2)
                s = _sigmoid_from_z(z)
                sp = sp + s
                d = s - spv
                sm = sm + d * d
                spv = s
            return (t1, t2, t3, sp, sm, spv)

        t1, t2, t3, sparse_acc, smooth_acc, s_prev = _row_loop

        sp, sm, spv = sparse_acc, smooth_acc, s_prev
        for ti in range(_MAIN, _SEG):
            raw = b_vec + ti
            valid = raw < _N
            x = plsc.load_gather(rbuf, [jnp.minimum(raw, _N - 1)])
            z = -x
            zm = jnp.where(valid, z, pos_inf)
            m1 = jnp.maximum(t1, zm)
            t1 = jnp.minimum(t1, zm)
            m2 = jnp.maximum(t2, m1)
            t2 = jnp.minimum(t2, m1)
            t3 = jnp.minimum(t3, m2)
            s = _sigmoid_from_z(z)
            sp = sp + jnp.where(valid, s, zeros)
            d = s - spv
            sm = sm + jnp.where(valid, d * d, zeros)
            spv = s
        sparse_acc, smooth_acc = sp, sm

        gsum = jnp.float32(0.0)
        for _ in range(3):
            g = jnp.min(t1)
            gsum = gsum + g
            gv = jnp.full((_L,), g)
            hit = lanes == plsc.all_reduce_ffs(t1 == gv)
            t1 = jnp.where(hit, t2, t1)
            t2 = jnp.where(hit, t3, t2)
            t3 = jnp.where(hit, pos_inf, t3)
        bag = gsum * _BAG_SCALE
        bag_acc = jnp.where(lanes == j, bag, bag_acc)

    label_copy.wait()
    y = plsc.load_gather(label_v, [row0 + jnp.minimum(lanes, _RPW - 1)])
    b = bag_acc
    ce_vec = jnp.maximum(b, 0.0) - b * y + _log1p_newton(jnp.exp(-jnp.abs(b)))
    ce_vec = jnp.where(lanes < _RPW, ce_vec, zeros)

    ce_p = jnp.sum(ce_vec) * (1.0 / _B)
    sm_p = jnp.sum(smooth_acc) * (1.0 / (_B * (_N - 1)))
    sp_p = jnp.sum(sparse_acc) * (1.0 / (_B * _N))
    tot_p = ce_p + _SMOOTH_W * sm_p + _SPARSE_W * sp_p

    res = jnp.where(lanes == 0, tot_p,
          jnp.where(lanes == 1, ce_p,
          jnp.where(lanes == 2, sm_p,
          jnp.where(lanes == 3, sp_p, zeros))))
    out_v[...] = res
    pltpu.sync_copy(out_v, out_hbm.at[wid])


@jax.jit
def _run(logits, label):
    out = pl.kernel(
        _body,
        out_type=jax.ShapeDtypeStruct((_NW, _L), jnp.float32),
        mesh=plsc.VectorSubcoreMesh(core_axis_name="c", subcore_axis_name="s"),
        compiler_params=pltpu.CompilerParams(needs_layout_passes=False),
        scratch_types=[
            pltpu.VMEM((_N,), jnp.float32),
            pltpu.VMEM((_N,), jnp.float32),
            pltpu.VMEM((_B,), jnp.float32),
            pltpu.VMEM((_L,), jnp.float32),
            pltpu.SemaphoreType.DMA,
            pltpu.SemaphoreType.DMA,
            pltpu.SemaphoreType.DMA,
        ],
    )(logits, label.astype(jnp.float32))
    s = out.sum(axis=0)
    return (s[0], s[1], s[2], s[3])


def kernel(logits, label):
    return _run(logits, label)


# speedup vs baseline: 1.1403x; 1.1403x over previous
import functools

import jax
import jax.numpy as jnp
from jax import lax
from jax.experimental import pallas as pl
from jax.experimental.pallas import tpu as pltpu
from jax.experimental.pallas import tpu_sc as plsc

_SMOOTH_W = 0.0008
_SPARSE_W = 0.0008

_L = 16
_NC = 2
_NS = 16
_NW = _NC * _NS
_B = 128
_N = 32768
_RPW = _B // _NW
_UNROLL = 8

_BAG_SCALE = -1.0 / 3.0

_TC_ROWS = 8


def _log1p_newton(z):
    w = 1.0 + z
    t = z * (1.0 - z * (0.5 - z * (1.0 / 3.0)))
    for _ in range(3):
        t = t - 1.0 + w * jnp.exp(-t)
    return t


def _sc_body(logits_hbm, label_hbm, out_hbm, buf0, buf1, label_v, out_v,
             sem0, sem1, lsem):
    cid = lax.axis_index("c")
    sid = lax.axis_index("s")
    wid = sid * _NC + cid
    lanes = lax.iota(jnp.int32, _L)

    bufs = [buf0, buf1]
    sems = [sem0, sem1]
    row0 = wid * _RPW
    copies = [pltpu.async_copy(logits_hbm.at[row0], bufs[0], sems[0])]
    label_copy = pltpu.async_copy(label_hbm, label_v, lsem)

    pos_inf = jnp.full((_L,), jnp.inf, dtype=jnp.float32)
    zeros = jnp.zeros((_L,), jnp.float32)
    bag_acc = zeros

    for j in range(_RPW):
        if j + 1 < _RPW:
            copies.append(pltpu.async_copy(
                logits_hbm.at[row0 + j + 1], bufs[(j + 1) % 2],
                sems[(j + 1) % 2]))
        copies[j].wait()
        rbuf = bufs[j % 2]

        @plsc.parallel_loop(
            0, _N, step=_L, unroll=_UNROLL,
            carry=(pos_inf, pos_inf, pos_inf))
        def _row_loop(i, carry, rbuf=rbuf):
            t1, t2, t3 = carry
            z = -rbuf[pl.ds(i, _L)]
            m1 = jnp.maximum(t1, z)
            t1 = jnp.minimum(t1, z)
            m2 = jnp.maximum(t2, m1)
            t2 = jnp.minimum(t2, m1)
            t3 = jnp.minimum(t3, m2)
            return (t1, t2, t3)

        t1, t2, t3 = _row_loop

        gsum = jnp.float32(0.0)
        for _ in range(3):
            g = jnp.min(t1)
            gsum = gsum + g
            gv = jnp.full((_L,), g)
            hit = lanes == plsc.all_reduce_ffs(t1 == gv)
            t1 = jnp.where(hit, t2, t1)
            t2 = jnp.where(hit, t3, t2)
            t3 = jnp.where(hit, pos_inf, t3)
        bag = gsum * _BAG_SCALE
        bag_acc = jnp.where(lanes == j, bag, bag_acc)

    label_copy.wait()
    y = plsc.load_gather(label_v, [row0 + jnp.minimum(lanes, _RPW - 1)])
    b = bag_acc
    ce_vec = jnp.maximum(b, 0.0) - b * y + _log1p_newton(jnp.exp(-jnp.abs(b)))
    ce_vec = jnp.where(lanes < _RPW, ce_vec, zeros)
    ce_p = jnp.sum(ce_vec) * (1.0 / _B)

    out_v[...] = jnp.where(lanes == 0, ce_p, zeros)
    pltpu.sync_copy(out_v, out_hbm.at[wid])


def _sc_topk_ce(logits, label):
    return pl.kernel(
        _sc_body,
        out_type=jax.ShapeDtypeStruct((_NW, _L), jnp.float32),
        mesh=plsc.VectorSubcoreMesh(core_axis_name="c", subcore_axis_name="s"),
        compiler_params=pltpu.CompilerParams(needs_layout_passes=False),
        scratch_types=[
            pltpu.VMEM((_N,), jnp.float32),
            pltpu.VMEM((_N,), jnp.float32),
            pltpu.VMEM((_B,), jnp.float32),
            pltpu.VMEM((_L,), jnp.float32),
            pltpu.SemaphoreType.DMA,
            pltpu.SemaphoreType.DMA,
            pltpu.SemaphoreType.DMA,
        ],
    )(logits, label)


def _tc_body(x_ref, o_ref):
    i = pl.program_id(0)
    x = x_ref[...]
    s = 1.0 / (1.0 + jnp.exp(-x))
    d = s[:, 1:] - s[:, :-1]
    sm = jnp.sum(d * d)
    sp = jnp.sum(s)

    @pl.when(i == 0)
    def _():
        o_ref[0] = 0.0
        o_ref[1] = 0.0

    o_ref[0] = o_ref[0] + sm
    o_ref[1] = o_ref[1] + sp


def _tc_sums(logits):
    return pl.pallas_call(
        _tc_body,
        grid=(_B // _TC_ROWS,),
        in_specs=[pl.BlockSpec((_TC_ROWS, _N), lambda i: (i, 0))],
        out_specs=pl.BlockSpec(memory_space=pltpu.SMEM),
        out_shape=jax.ShapeDtypeStruct((2,), jnp.float32),
    )(logits)


@jax.jit
def _run(logits, label):
    sc_out = _sc_topk_ce(logits, label.astype(jnp.float32))
    tc_out = _tc_sums(logits)
    ce = jnp.sum(sc_out[:, 0])
    smooth = tc_out[0] * (1.0 / (_B * (_N - 1)))
    sparse = tc_out[1] * (1.0 / (_B * _N))
    total = ce + _SMOOTH_W * smooth + _SPARSE_W * sparse
    return (total, ce, smooth, sparse)


def kernel(logits, label):
    return _run(logits, label)
